# baseline (speedup 1.0000x reference)
import functools

import jax
import jax.numpy as jnp
from jax import lax
from jax.experimental import pallas as pl
from jax.experimental.pallas import tpu as pltpu
from jax.experimental.pallas import tpu_sc as plsc

MARGIN = 1.0
PADD_IDX = 0

_B = 512


def _sc_hist_gather(tval_hbm, w_hbm,
                    tv_v, ones_v, zeros_v, shared, sem):
    NC, NS = 2, 16
    c = lax.axis_index("c")
    s = lax.axis_index("s")
    g = c * NS + s
    n_per = tv_v.shape[0]
    base = g * n_per

    @pl.when(s == 0)
    def _zero_shared():
        for i in range(zeros_v.shape[0] // 16):
            zeros_v[pl.ds(i * 16, 16)] = jnp.zeros((16,), jnp.float32)
        pltpu.sync_copy(zeros_v, shared)

    for i in range(n_per // 16):
        ones_v[pl.ds(i * 16, 16)] = jnp.full((16,), 1.0, jnp.float32)
    pltpu.sync_copy(tval_hbm.at[pl.ds(base, n_per)], tv_v)
    plsc.subcore_barrier()
    pltpu.sync_copy(ones_v, shared.at[tv_v], add=True)
    plsc.subcore_barrier()

    @pl.when(s == 0)
    def _write_w():
        pltpu.sync_copy(shared, w_hbm.at[c])


def _tc_body(tcol_blk_ref, w2_ref, preds_ref, out_ref,
             tot_ref, cnt_ref):
    k = pl.program_id(1)
    nk = pl.num_programs(1)
    V = preds_ref.shape[-1]

    @pl.when(k == 0)
    def _init():
        tot_ref[0, 0] = 0.0
        cnt_ref[0, 0] = 0.0

    rows = preds_ref[0]
    t_blk = tcol_blk_ref[...]
    iota_bv = lax.broadcasted_iota(jnp.int32, rows.shape, 1)
    sel = jnp.where(iota_bv == t_blk, rows, 0.0)
    ones_col = jnp.ones((V, 1), jnp.float32)
    d_blk = lax.dot_general(
        sel, ones_col, (((1,), (0,)), ((), ())),
        preferred_element_type=jnp.float32)
    relu = jnp.maximum(rows + (MARGIN - d_blk), 0.0)
    col = lax.broadcasted_iota(jnp.int32, (1, V), 1)
    w = jnp.where(col == PADD_IDX, 0.0, w2_ref[0:1, :] + w2_ref[1:2, :])
    row_sums = lax.dot_general(
        relu, w, (((1,), (1,)), ((), ())),
        preferred_element_type=jnp.float32)
    mask = (t_blk != PADD_IDX).astype(jnp.float32)
    tot_ref[0, 0] += jnp.sum(row_sums * mask)
    cnt_ref[0, 0] += jnp.sum(mask)

    @pl.when(k == nk - 1)
    def _fini():
        out_ref[...] = jnp.concatenate(
            [jnp.full((1, 1, 1), tot_ref[0, 0], jnp.float32),
             jnp.full((1, 1, 1), cnt_ref[0, 0], jnp.float32)], axis=2)


def kernel(preds, targets):
    Bt, T1, V = preds.shape
    T = T1 - 1
    N = Bt * T
    t32 = targets.astype(jnp.int32).reshape(N)

    n_per = N // 32
    sc = pl.kernel(
        _sc_hist_gather,
        mesh=plsc.VectorSubcoreMesh(core_axis_name="c", subcore_axis_name="s"),
        out_type=[
            jax.ShapeDtypeStruct((2, V), jnp.float32),
        ],
        scratch_types=[
            pltpu.VMEM((n_per,), jnp.int32),
            pltpu.VMEM((n_per,), jnp.float32),
            pltpu.VMEM((V,), jnp.float32),
            pltpu.VMEM_SHARED((V,), jnp.float32),
            pltpu.SemaphoreType.DMA,
        ],
    )
    (w2,) = sc(t32)

    tcol = t32.reshape(N, 1)
    nk = T // _B
    out = pl.pallas_call(
        _tc_body,
        grid=(Bt, nk),
        in_specs=[
            pl.BlockSpec((_B, 1), lambda b, k: (b * nk + k, b - b)),
            pl.BlockSpec((2, V), lambda b, k: (b - b, k - k)),
            pl.BlockSpec((1, _B, V), lambda b, k: (b, k, b - b)),
        ],
        out_specs=pl.BlockSpec((1, 1, 2), lambda b, k: (b, k - k, k - k)),
        out_shape=jax.ShapeDtypeStruct((Bt, 1, 2), jnp.float32),
        scratch_shapes=[
            pltpu.SMEM((1, 1), jnp.float32),
            pltpu.SMEM((1, 1), jnp.float32),
        ],
        compiler_params=pltpu.CompilerParams(
            dimension_semantics=("parallel", "arbitrary")),
    )(tcol, w2, preds)
    tot = out[0, 0, 0] + out[1, 0, 0]
    cnt = out[0, 0, 1] + out[1, 0, 1]
    return tot / (cnt * cnt)


# speedup vs baseline: 1.0669x; 1.0669x over previous
import functools

import jax
import jax.numpy as jnp
from jax import lax
from jax.experimental import pallas as pl
from jax.experimental.pallas import tpu as pltpu
from jax.experimental.pallas import tpu_sc as plsc

MARGIN = 1.0
PADD_IDX = 0

_B = 1024


def _sc_hist_gather(tval_hbm, w_hbm,
                    tv_v, ones_v, zeros_v, shared, sem):
    NC, NS = 2, 16
    c = lax.axis_index("c")
    s = lax.axis_index("s")
    g = c * NS + s
    n_per = tv_v.shape[0]
    base = g * n_per

    @pl.when(s == 0)
    def _zero_shared():
        for i in range(zeros_v.shape[0] // 16):
            zeros_v[pl.ds(i * 16, 16)] = jnp.zeros((16,), jnp.float32)
        pltpu.sync_copy(zeros_v, shared)

    for i in range(n_per // 16):
        ones_v[pl.ds(i * 16, 16)] = jnp.full((16,), 1.0, jnp.float32)
    pltpu.sync_copy(tval_hbm.at[pl.ds(base, n_per)], tv_v)
    plsc.subcore_barrier()
    pltpu.sync_copy(ones_v, shared.at[tv_v], add=True)
    plsc.subcore_barrier()

    @pl.when(s == 0)
    def _write_w():
        pltpu.sync_copy(shared, w_hbm.at[c])


def _tc_body(tcol_blk_ref, w2_ref, preds_ref, out_ref,
             tot_ref, cnt_ref):
    b = pl.program_id(0)
    k = pl.program_id(1)
    nb = pl.num_programs(0)
    nk = pl.num_programs(1)
    V = preds_ref.shape[-1]

    @pl.when(jnp.logical_and(b == 0, k == 0))
    def _init():
        tot_ref[0, 0] = 0.0
        cnt_ref[0, 0] = 0.0

    rows = preds_ref[0]
    t_blk = tcol_blk_ref[...]
    iota_bv = lax.broadcasted_iota(jnp.int32, rows.shape, 1)
    sel = jnp.where(iota_bv == t_blk, rows, 0.0)
    ones_col = jnp.ones((V, 1), jnp.float32)
    d_blk = lax.dot_general(
        sel, ones_col, (((1,), (0,)), ((), ())),
        preferred_element_type=jnp.float32)
    relu = jnp.maximum(rows + (MARGIN - d_blk), 0.0)
    col = lax.broadcasted_iota(jnp.int32, (1, V), 1)
    w = jnp.where(col == PADD_IDX, 0.0, w2_ref[0:1, :] + w2_ref[1:2, :])
    row_sums = lax.dot_general(
        relu, w, (((1,), (1,)), ((), ())),
        preferred_element_type=jnp.float32)
    mask = (t_blk != PADD_IDX).astype(jnp.float32)
    tot_ref[0, 0] += jnp.sum(row_sums * mask)
    cnt_ref[0, 0] += jnp.sum(mask)

    @pl.when(jnp.logical_and(b == nb - 1, k == nk - 1))
    def _fini():
        cnt = cnt_ref[0, 0]
        out_ref[...] = jnp.full((1, 1), tot_ref[0, 0] / (cnt * cnt),
                                jnp.float32)


def kernel(preds, targets):
    Bt, T1, V = preds.shape
    T = T1 - 1
    N = Bt * T
    t32 = targets.astype(jnp.int32).reshape(N)

    n_per = N // 32
    sc = pl.kernel(
        _sc_hist_gather,
        mesh=plsc.VectorSubcoreMesh(core_axis_name="c", subcore_axis_name="s"),
        out_type=[
            jax.ShapeDtypeStruct((2, V), jnp.float32),
        ],
        scratch_types=[
            pltpu.VMEM((n_per,), jnp.int32),
            pltpu.VMEM((n_per,), jnp.float32),
            pltpu.VMEM((V,), jnp.float32),
            pltpu.VMEM_SHARED((V,), jnp.float32),
            pltpu.SemaphoreType.DMA,
        ],
    )
    (w2,) = sc(t32)

    tcol = t32.reshape(N, 1)
    nk = T // _B
    out = pl.pallas_call(
        _tc_body,
        grid=(Bt, nk),
        in_specs=[
            pl.BlockSpec((_B, 1), lambda b, k: (b * nk + k, b - b)),
            pl.BlockSpec((2, V), lambda b, k: (b - b, k - k)),
            pl.BlockSpec((1, _B, V), lambda b, k: (b, k, b - b)),
        ],
        out_specs=pl.BlockSpec((1, 1), lambda b, k: (b - b, k - k)),
        out_shape=jax.ShapeDtypeStruct((1, 1), jnp.float32),
        scratch_shapes=[
            pltpu.SMEM((1, 1), jnp.float32),
            pltpu.SMEM((1, 1), jnp.float32),
        ],
        compiler_params=pltpu.CompilerParams(
            dimension_semantics=("arbitrary", "arbitrary")),
    )(tcol, w2, preds)
    return out.reshape(())


# speedup vs baseline: 1.0704x; 1.0032x over previous
import jax
import jax.numpy as jnp
from jax import lax
from jax.experimental import pallas as pl
from jax.experimental.pallas import tpu as pltpu
from jax.experimental.pallas import tpu_sc as plsc

MARGIN = 1.0
PADD_IDX = 0

_B = 1024


def _sc_hist(tval_hbm, w_hbm, tv_v, ones_v, zeros_v, shared):
    NS = 16
    c = lax.axis_index("c")
    s = lax.axis_index("s")
    g = c * NS + s
    n_per = tv_v.shape[0]
    base = g * n_per

    @pl.when(s == 0)
    def _zero_shared():
        for i in range(zeros_v.shape[0] // 16):
            zeros_v[pl.ds(i * 16, 16)] = jnp.zeros((16,), jnp.float32)
        pltpu.sync_copy(zeros_v, shared)

    for i in range(n_per // 16):
        ones_v[pl.ds(i * 16, 16)] = jnp.full((16,), 1.0, jnp.float32)
    pltpu.sync_copy(tval_hbm.at[pl.ds(base, n_per)], tv_v)
    plsc.subcore_barrier()
    pltpu.sync_copy(ones_v, shared.at[tv_v], add=True)
    plsc.subcore_barrier()

    @pl.when(s == 0)
    def _write_w():
        pltpu.sync_copy(shared, w_hbm.at[c])


def _tc_body(tcol_blk_ref, w2_ref, preds_ref, out_ref,
             tot_ref, cnt_ref):
    b = pl.program_id(0)
    k = pl.program_id(1)
    nb = pl.num_programs(0)
    nk = pl.num_programs(1)
    V = preds_ref.shape[-1]

    @pl.when(jnp.logical_and(b == 0, k == 0))
    def _init():
        tot_ref[0, 0] = 0.0
        cnt_ref[0, 0] = 0.0

    rows = preds_ref[0]
    t_blk = tcol_blk_ref[...]
    iota_bv = lax.broadcasted_iota(jnp.int32, rows.shape, 1)
    sel = jnp.where(iota_bv == t_blk, rows, 0.0)
    ones_col = jnp.ones((V, 1), jnp.float32)
    d_blk = lax.dot_general(
        sel, ones_col, (((1,), (0,)), ((), ())),
        preferred_element_type=jnp.float32)
    relu = jnp.maximum(rows + (MARGIN - d_blk), 0.0)
    col = lax.broadcasted_iota(jnp.int32, (1, V), 1)
    w = jnp.where(col == PADD_IDX, 0.0, w2_ref[0:1, :] + w2_ref[1:2, :])
    row_sums = lax.dot_general(
        relu, w, (((1,), (1,)), ((), ())),
        preferred_element_type=jnp.float32)
    mask = (t_blk != PADD_IDX).astype(jnp.float32)
    tot_ref[0, 0] += jnp.sum(row_sums * mask)
    cnt_ref[0, 0] += jnp.sum(mask)

    @pl.when(jnp.logical_and(b == nb - 1, k == nk - 1))
    def _fini():
        cnt = cnt_ref[0, 0]
        out_ref[...] = jnp.full((1, 1), tot_ref[0, 0] / (cnt * cnt),
                                jnp.float32)


def kernel(preds, targets):
    Bt, T1, V = preds.shape
    T = T1 - 1
    N = Bt * T
    t32 = targets.astype(jnp.int32).reshape(N)

    n_per = N // 32
    sc = pl.kernel(
        _sc_hist,
        mesh=plsc.VectorSubcoreMesh(core_axis_name="c", subcore_axis_name="s"),
        out_type=[
            jax.ShapeDtypeStruct((2, V), jnp.float32),
        ],
        scratch_types=[
            pltpu.VMEM((n_per,), jnp.int32),
            pltpu.VMEM((n_per,), jnp.float32),
            pltpu.VMEM((V,), jnp.float32),
            pltpu.VMEM_SHARED((V,), jnp.float32),
        ],
    )
    (w2,) = sc(t32)

    tcol = t32.reshape(N, 1)
    nk = T // _B
    out = pl.pallas_call(
        _tc_body,
        grid=(Bt, nk),
        in_specs=[
            pl.BlockSpec((_B, 1), lambda b, k: (b * nk + k, b - b)),
            pl.BlockSpec((2, V), lambda b, k: (b - b, k - k)),
            pl.BlockSpec((1, _B, V), lambda b, k: (b, k, b - b)),
        ],
        out_specs=pl.BlockSpec((1, 1), lambda b, k: (b - b, k - k)),
        out_shape=jax.ShapeDtypeStruct((1, 1), jnp.float32),
        scratch_shapes=[
            pltpu.SMEM((1, 1), jnp.float32),
            pltpu.SMEM((1, 1), jnp.float32),
        ],
        compiler_params=pltpu.CompilerParams(
            dimension_semantics=("arbitrary", "arbitrary")),
    )(tcol, w2, preds)
    return out.reshape(())
